# Initial kernel scaffold; baseline (speedup 1.0000x reference)
import jax
import jax.numpy as jnp
from jax.experimental import pallas as pl


def kernel(x):
    raise NotImplementedError("write your pallas kernel here")


# speedup vs baseline: 2.3774x; 2.3774x over previous
import functools

import jax
import jax.numpy as jnp
from jax import lax
from jax.experimental import pallas as pl
from jax.experimental.pallas import tpu as pltpu
from jax.experimental.pallas import tpu_sc as plsc

_K = 256
_ROWS = 128
_COLS = 8192
_L = 16
_CHUNKS = _COLS // _L
_NC = 2
_NS = 16
_NW = _NC * _NS
_RPW = _ROWS // _NW
_H = 4096


def kernel(x):
    mesh = plsc.VectorSubcoreMesh(core_axis_name="c", subcore_axis_name="s")

    @functools.partial(
        pl.kernel,
        mesh=mesh,
        out_type=jax.ShapeDtypeStruct((_ROWS, _COLS), jnp.float32),
        scratch_types=[
            pltpu.VMEM((_COLS,), jnp.float32),
            pltpu.VMEM((_H,), jnp.int32),
        ],
        compiler_params=pltpu.CompilerParams(needs_layout_passes=False),
    )
    def wta(x_hbm, out_hbm, xv, hist):
        zeros16 = jnp.zeros((_L,), jnp.int32)
        ones16 = jnp.ones((_L,), jnp.int32)
        wid = lax.axis_index("s") * _NC + lax.axis_index("c")

        def zero_chunk(i, carry):
            hist[pl.ds(i * _L, _L)] = zeros16
            return carry

        lax.fori_loop(0, _H // _L, zero_chunk, 0)

        def walk(nchunks, target):
            def body(i, carry):
                tot, found, h, cab = carry
                base = (nchunks - 1 - i) * _L
                cvec = hist[pl.ds(base, _L)]
                hist[pl.ds(base, _L)] = zeros16
                rc = lax.rev(cvec, (0,))
                cum = tot + plsc.cumsum(rc)
                ge = cum >= target
                csum = jnp.sum(cvec)
                popc = jnp.max(plsc.all_reduce_population_count(ge))
                hit = jnp.logical_and(found == 0, popc > 0)
                h_new = base + popc - 1
                cab_new = tot + jnp.sum(jnp.where(ge, 0, rc))
                h = jnp.where(hit, h_new, h)
                cab = jnp.where(hit, cab_new, cab)
                found = jnp.where(hit, 1, found)
                return (tot + csum, found, h, cab)

            z = jnp.int32(0)
            _, _, h, cab = lax.fori_loop(0, nchunks, body, (z, z, z, z))
            return h, cab

        def keys(i):
            xb = xv[pl.ds(i * _L, _L)]
            bits = lax.bitcast_convert_type(xb, jnp.int32)
            return xb, bits ^ ((bits >> 31) & jnp.int32(0x7FFFFFFF))

        def do_row(j, carry):
            row = wid * _RPW + j
            pltpu.sync_copy(x_hbm.at[row], xv)

            def p1(i, c):
                _, sk = keys(i)
                plsc.addupdate_scatter(hist, [(sk >> 20) + 2048], ones16)
                return c

            lax.fori_loop(0, _CHUNKS, p1, 0)
            h1, cab1 = walk(_H // _L, jnp.int32(_K))
            pfx1 = h1 - 2048
            r1 = _K - cab1

            def p2(i, c):
                _, sk = keys(i)
                m = (sk >> 20) == pfx1
                plsc.addupdate_scatter(hist, [(sk >> 8) & 0xFFF], ones16,
                                       mask=m)
                return c

            lax.fori_loop(0, _CHUNKS, p2, 0)
            b2, cab2 = walk(_H // _L, r1)
            pfx2 = (pfx1 << 12) | b2
            r2 = r1 - cab2

            def p3(i, c):
                _, sk = keys(i)
                m = (sk >> 8) == pfx2
                plsc.addupdate_scatter(hist, [sk & 0xFF], ones16, mask=m)
                return c

            lax.fori_loop(0, _CHUNKS, p3, 0)
            b3, cab3 = walk(256 // _L, r2)
            t = (pfx2 << 8) | b3
            tie_budget = r2 - cab3

            def pout(i, used):
                xb, sk = keys(i)
                eq = sk == t
                eq_i = jnp.where(eq, 1, 0)
                tie_rank = used + plsc.cumsum(eq_i)
                keep = (sk > t) | (eq & (tie_rank <= tie_budget))
                xv[pl.ds(i * _L, _L)] = jnp.where(keep, xb, 0.0)
                return used + jnp.sum(eq_i)

            lax.fori_loop(0, _CHUNKS, pout, jnp.int32(0))
            pltpu.sync_copy(xv, out_hbm.at[row])
            return carry

        lax.fori_loop(0, _RPW, do_row, 0)

    return wta(x)


# speedup vs baseline: 4.7354x; 1.9918x over previous
import functools

import jax
import jax.numpy as jnp
from jax import lax
from jax.experimental import pallas as pl
from jax.experimental.pallas import tpu as pltpu
from jax.experimental.pallas import tpu_sc as plsc

_K = 256
_ROWS = 128
_COLS = 8192
_L = 16
_CHUNKS = _COLS // _L
_NC = 2
_NS = 16
_NW = _NC * _NS
_RPW = _ROWS // _NW
_H = 4096
_UNROLL = 4


def kernel(x):
    mesh = plsc.VectorSubcoreMesh(core_axis_name="c", subcore_axis_name="s")

    @functools.partial(
        pl.kernel,
        mesh=mesh,
        out_type=jax.ShapeDtypeStruct((_ROWS, _COLS), jnp.float32),
        scratch_types=[
            pltpu.VMEM((_COLS,), jnp.float32),
            pltpu.VMEM((_COLS,), jnp.int32),
            pltpu.VMEM((_H,), jnp.int32),
            pltpu.VMEM((_COLS + _L,), jnp.int32),
        ],
        compiler_params=pltpu.CompilerParams(needs_layout_passes=False),
    )
    def wta(x_hbm, out_hbm, xv, skv, hist, cand):
        zeros16 = jnp.zeros((_L,), jnp.int32)
        ones16 = jnp.ones((_L,), jnp.int32)
        lanes = lax.iota(jnp.int32, _L)
        wid = lax.axis_index("s") * _NC + lax.axis_index("c")

        def zero_chunk(i, carry):
            hist[pl.ds(i * _L, _L)] = zeros16
            return carry

        lax.fori_loop(0, _H // _L, zero_chunk, 0)

        def walk(nchunks, target):
            def step(base, carry):
                tot, found, h, cab, cat = carry
                cvec = hist[pl.ds(base, _L)]
                hist[pl.ds(base, _L)] = zeros16
                rc = lax.rev(cvec, (0,))
                cum = tot + plsc.cumsum(rc)
                ge = cum >= target
                ge2 = (cum - rc) >= target
                csum = jnp.sum(cvec)
                popc = jnp.max(plsc.all_reduce_population_count(ge))
                hit = jnp.logical_and(found == 0, popc > 0)
                h_new = base + popc - 1
                cab_new = tot + jnp.sum(jnp.where(ge, 0, rc))
                cat_new = jnp.sum(jnp.where(ge, rc, 0)) - \
                    jnp.sum(jnp.where(ge2, rc, 0))
                h = jnp.where(hit, h_new, h)
                cab = jnp.where(hit, cab_new, cab)
                cat = jnp.where(hit, cat_new, cat)
                found = jnp.where(hit, 1, found)
                return (tot + csum, found, h, cab, cat)

            def body(i, carry):
                for u in range(_UNROLL):
                    carry = step((nchunks - 1 - (i * _UNROLL + u)) * _L,
                                 carry)
                return carry

            z = jnp.int32(0)
            carry = (z, z, z, z, z)
            n_un, n_rem = divmod(nchunks, _UNROLL)
            carry = lax.fori_loop(0, n_un, body, carry)
            for u in range(n_rem):
                carry = step((n_rem - 1 - u) * _L, carry)
            _, _, h, cab, cat = carry
            return h, cab, cat

        def key_chunk(i):
            xb = xv[pl.ds(i * _L, _L)]
            bits = lax.bitcast_convert_type(xb, jnp.int32)
            return xb, bits ^ ((bits >> 31) & jnp.int32(0x7FFFFFFF))

        def do_row(j, carry):
            row = wid * _RPW + j
            pltpu.sync_copy(x_hbm.at[row], xv)

            def p1(i, c):
                for u in range(_UNROLL):
                    ch = i * _UNROLL + u
                    _, sk = key_chunk(ch)
                    skv[pl.ds(ch * _L, _L)] = sk
                    plsc.addupdate_scatter(hist, [(sk >> 20) + 2048], ones16)
                return c

            lax.fori_loop(0, _CHUNKS // _UNROLL, p1, 0)
            h1, cab1, _ = walk(_H // _L, jnp.int32(_K))
            pfx1 = h1 - 2048
            r1 = _K - cab1

            def pk(i, off):
                for u in range(_UNROLL):
                    ch = i * _UNROLL + u
                    sk = skv[pl.ds(ch * _L, _L)]
                    m = (sk >> 20) == pfx1
                    plsc.store_compressed(cand.at[pl.ds(off, _L)], sk, mask=m)
                    off = off + jnp.max(plsc.all_reduce_population_count(m))
                return off

            cnt = lax.fori_loop(0, _CHUNKS // _UNROLL, pk, jnp.int32(0))
            nch = (cnt + _L - 1) // _L

            def refine(pshift, pfx, bshift, bmask, nbins, target):
                def body(i, c):
                    base = i * _L
                    sk = cand[pl.ds(base, _L)]
                    m = jnp.logical_and(lanes < (cnt - base),
                                        (sk >> pshift) == pfx)
                    plsc.addupdate_scatter(hist, [(sk >> bshift) & bmask],
                                           ones16, mask=m)
                    return c

                lax.fori_loop(0, nch, body, 0)
                return walk(nbins // _L, target)

            b2a, cabA, _ = refine(20, pfx1, 12, 0xFF, 256, r1)
            pfx_a = (pfx1 << 8) | b2a
            r2a = r1 - cabA
            b2b, cabB, _ = refine(12, pfx_a, 4, 0xFF, 256, r2a)
            pfx_b = (pfx_a << 8) | b2b
            r2b = r2a - cabB
            b3, cabC, cnt_at = refine(4, pfx_b, 0, 0xF, 16, r2b)
            t = (pfx_b << 4) | b3
            tie_budget = r2b - cabC

            def pout_fast(_):
                def body(i, c):
                    for u in range(_UNROLL):
                        ch = i * _UNROLL + u
                        xb = xv[pl.ds(ch * _L, _L)]
                        sk = skv[pl.ds(ch * _L, _L)]
                        xv[pl.ds(ch * _L, _L)] = jnp.where(sk >= t, xb, 0.0)
                    return c

                lax.fori_loop(0, _CHUNKS // _UNROLL, body, 0)
                return 0

            def pout_exact(_):
                def body(i, used):
                    xb = xv[pl.ds(i * _L, _L)]
                    sk = skv[pl.ds(i * _L, _L)]
                    eq = sk == t
                    eq_i = jnp.where(eq, 1, 0)
                    tie_rank = used + plsc.cumsum(eq_i)
                    keep = (sk > t) | jnp.logical_and(eq,
                                                      tie_rank <= tie_budget)
                    xv[pl.ds(i * _L, _L)] = jnp.where(keep, xb, 0.0)
                    return used + jnp.sum(eq_i)

                lax.fori_loop(0, _CHUNKS, body, jnp.int32(0))
                return 0

            lax.cond(tie_budget == cnt_at, pout_fast, pout_exact, 0)
            pltpu.sync_copy(xv, out_hbm.at[row])
            return carry

        lax.fori_loop(0, _RPW, do_row, 0)

    return wta(x)


# speedup vs baseline: 5.3369x; 1.1270x over previous
import functools

import jax
import jax.numpy as jnp
from jax import lax
from jax.experimental import pallas as pl
from jax.experimental.pallas import tpu as pltpu
from jax.experimental.pallas import tpu_sc as plsc

_K = 256
_ROWS = 128
_COLS = 8192
_L = 16
_CHUNKS = _COLS // _L
_NC = 2
_NS = 16
_NW = _NC * _NS
_RPW = _ROWS // _NW
_H = 4096
_UN = 4


def kernel(x):
    mesh = plsc.VectorSubcoreMesh(core_axis_name="c", subcore_axis_name="s")

    @functools.partial(
        pl.kernel,
        mesh=mesh,
        out_type=jax.ShapeDtypeStruct((_ROWS, _COLS), jnp.float32),
        scratch_types=[
            pltpu.VMEM((_COLS,), jnp.float32),
            pltpu.VMEM((_H,), jnp.int32),
            pltpu.VMEM((_COLS + _L,), jnp.int32),
            pltpu.VMEM((_COLS + _L,), jnp.int32),
            pltpu.VMEM((_COLS,), jnp.float32),
            pltpu.VMEM((_K + _L,), jnp.int32),
            pltpu.VMEM((_K + _L,), jnp.int32),
        ],
        compiler_params=pltpu.CompilerParams(needs_layout_passes=False),
    )
    def wta(x_hbm, out_hbm, xv, hist, csk, cix, outv, ksk, kix):
        zi16 = jnp.zeros((_L,), jnp.int32)
        zf16 = jnp.zeros((_L,), jnp.float32)
        ones16 = jnp.ones((_L,), jnp.int32)
        lanes = lax.iota(jnp.int32, _L)
        wid = lax.axis_index("s") * _NC + lax.axis_index("c")

        def zero_init(i, carry):
            hist[pl.ds(i * _L, _L)] = zi16
            outv[pl.ds(i * _L, _L)] = zf16
            outv[pl.ds((i + _H // _L) * _L, _L)] = zf16
            return carry

        lax.fori_loop(0, _H // _L, zero_init, 0)

        def sortkey(bits):
            return bits ^ ((bits >> 31) & jnp.int32(0x7FFFFFFF))

        def walk(nchunks, target):
            def step(base, carry):
                tot, found, sv, sbase, stot = carry
                cvec = hist[pl.ds(base, _L)]
                hist[pl.ds(base, _L)] = zi16
                csum = jnp.sum(cvec)
                hit = jnp.logical_and(found == 0, tot + csum >= target)
                sv = jnp.where(hit, cvec, sv)
                sbase = jnp.where(hit, base, sbase)
                stot = jnp.where(hit, tot, stot)
                found = jnp.where(hit, 1, found)
                return (tot + csum, found, sv, sbase, stot)

            def body(i, carry):
                for u in range(_UN):
                    carry = step((nchunks - 1 - (i * _UN + u)) * _L, carry)
                return carry

            z = jnp.int32(0)
            carry = (z, z, zi16, z, z)
            n_un, n_rem = divmod(nchunks, _UN)
            carry = lax.fori_loop(0, n_un, body, carry)
            for u in range(n_rem):
                carry = step((n_rem - 1 - u) * _L, carry)
            _, _, sv, sbase, stot = carry

            rc = lax.rev(sv, (0,))
            cum = stot + plsc.cumsum(rc)
            ge = cum >= target
            ge2 = (cum - rc) >= target
            popc = jnp.max(plsc.all_reduce_population_count(ge))
            h = sbase + popc - 1
            cab = stot + jnp.sum(jnp.where(ge, 0, rc))
            cnt_at = jnp.sum(jnp.where(ge, rc, 0)) - \
                jnp.sum(jnp.where(ge2, rc, 0))
            return h, cab, cnt_at

        def do_row(j, carry):
            row = wid * _RPW + j
            pltpu.sync_copy(x_hbm.at[row], xv)

            def p1(i, c):
                for u in range(_UN * 2):
                    ch = i * _UN * 2 + u
                    xb = xv[pl.ds(ch * _L, _L)]
                    sk = sortkey(lax.bitcast_convert_type(xb, jnp.int32))
                    plsc.addupdate_scatter(hist, [(sk >> 20) + 2048], ones16)
                return c

            lax.fori_loop(0, _CHUNKS // (_UN * 2), p1, 0)
            h1, cab1, _ = walk(_H // _L, jnp.int32(_K))
            pfx1 = h1 - 2048
            r1 = _K - cab1

            def pk(i, off):
                for u in range(_UN):
                    ch = i * _UN + u
                    xb = xv[pl.ds(ch * _L, _L)]
                    sk = sortkey(lax.bitcast_convert_type(xb, jnp.int32))
                    m = (sk >> 20) >= pfx1
                    plsc.store_compressed(csk.at[pl.ds(off, _L)], sk, mask=m)
                    plsc.store_compressed(cix.at[pl.ds(off, _L)],
                                          lanes + ch * _L, mask=m)
                    off = off + jnp.max(plsc.all_reduce_population_count(m))
                return off

            cnt = lax.fori_loop(0, _CHUNKS // _UN, pk, jnp.int32(0))
            nch = (cnt + _L - 1) // _L

            def refine(pshift, pfx, bshift, bmask, nbins, target):
                def body(i, c):
                    base = i * _L
                    sk = csk[pl.ds(base, _L)]
                    m = jnp.logical_and(lanes < (cnt - base),
                                        (sk >> pshift) == pfx)
                    plsc.addupdate_scatter(hist, [(sk >> bshift) & bmask],
                                           ones16, mask=m)
                    return c

                lax.fori_loop(0, nch, body, 0)
                return walk(nbins // _L, target)

            b2a, cabA, _ = refine(20, pfx1, 12, 0xFF, 256, r1)
            pfx_a = (pfx1 << 8) | b2a
            r2a = r1 - cabA
            b2b, cabB, _ = refine(12, pfx_a, 4, 0xFF, 256, r2a)
            pfx_b = (pfx_a << 8) | b2b
            r2b = r2a - cabB
            b3, cabC, cnt_at = refine(4, pfx_b, 0, 0xF, 16, r2b)
            t = (pfx_b << 4) | b3
            tie_budget = r2b - cabC

            def sel(i, c):
                off, used = c
                base = i * _L
                sk = csk[pl.ds(base, _L)]
                ix = cix[pl.ds(base, _L)]
                valid = lanes < (cnt - base)
                gt = jnp.logical_and(valid, sk > t)
                eq = jnp.logical_and(valid, sk == t)
                eq_i = jnp.where(eq, 1, 0)
                tie_rank = used + plsc.cumsum(eq_i)
                keep = gt | jnp.logical_and(eq, tie_rank <= tie_budget)
                plsc.store_compressed(ksk.at[pl.ds(off, _L)], sk, mask=keep)
                plsc.store_compressed(kix.at[pl.ds(off, _L)], ix, mask=keep)
                off = off + jnp.max(plsc.all_reduce_population_count(keep))
                return (off, jnp.max(tie_rank))

            lax.fori_loop(0, nch, sel, (jnp.int32(0), jnp.int32(0)))

            def emit(i, c):
                sk = ksk[pl.ds(i * _L, _L)]
                ix = kix[pl.ds(i * _L, _L)]
                vals = lax.bitcast_convert_type(sortkey(sk), jnp.float32)
                plsc.store_scatter(outv, [ix], vals)
                return c

            lax.fori_loop(0, _K // _L, emit, 0)
            pltpu.sync_copy(outv, out_hbm.at[row])

            def unemit(i, c):
                ix = kix[pl.ds(i * _L, _L)]
                plsc.store_scatter(outv, [ix], zf16)
                return c

            lax.fori_loop(0, _K // _L, unemit, 0)
            return carry

        lax.fori_loop(0, _RPW, do_row, 0)

    return wta(x)


# speedup vs baseline: 10.7694x; 2.0179x over previous
import functools

import jax
import jax.numpy as jnp
from jax import lax
from jax.experimental import pallas as pl
from jax.experimental.pallas import tpu as pltpu
from jax.experimental.pallas import tpu_sc as plsc

_K = 256
_ROWS = 128
_COLS = 8192
_L = 16
_CHUNKS = _COLS // _L
_NC = 2
_NS = 16
_NW = _NC * _NS
_RPW = _ROWS // _NW
_H = 4096


def kernel(x):
    mesh = plsc.VectorSubcoreMesh(core_axis_name="c", subcore_axis_name="s")

    @functools.partial(
        pl.kernel,
        mesh=mesh,
        out_type=jax.ShapeDtypeStruct((_ROWS, _COLS), jnp.float32),
        scratch_types=[
            pltpu.VMEM((_COLS,), jnp.float32),
            pltpu.VMEM((_H,), jnp.int32),
            pltpu.VMEM((_COLS + _L,), jnp.int32),
            pltpu.VMEM((_COLS + _L,), jnp.int32),
            pltpu.VMEM((_COLS,), jnp.float32),
            pltpu.VMEM((_K + _L,), jnp.int32),
            pltpu.VMEM((_K + _L,), jnp.int32),
        ],
        compiler_params=pltpu.CompilerParams(needs_layout_passes=False),
    )
    def wta(x_hbm, out_hbm, xv, hist, csk, cix, outv, ksk, kix):
        zi16 = jnp.zeros((_L,), jnp.int32)
        zf16 = jnp.zeros((_L,), jnp.float32)
        ones16 = jnp.ones((_L,), jnp.int32)
        lanes = lax.iota(jnp.int32, _L)
        wid = lax.axis_index("s") * _NC + lax.axis_index("c")

        @plsc.parallel_loop(0, _H // _L, unroll=8)
        def _zero_init(i):
            hist[pl.ds(i * _L, _L)] = zi16
            outv[pl.ds(i * _L, _L)] = zf16
            outv[pl.ds((i + _H // _L) * _L, _L)] = zf16

        def sortkey(bits):
            return bits ^ ((bits >> 31) & jnp.int32(0x7FFFFFFF))

        def walk(nchunks, target):
            z = jnp.int32(0)

            @plsc.parallel_loop(0, nchunks, unroll=4 if nchunks >= 4 else 1,
                                carry=(z, z, zi16, z, z))
            def ph1(i, carry):
                tot, found, sv, sbase, stot = carry
                base = (nchunks - 1 - i) * _L
                cvec = hist[pl.ds(base, _L)]
                hist[pl.ds(base, _L)] = zi16
                csum = jnp.sum(cvec)
                hit = jnp.logical_and(found == 0, tot + csum >= target)
                sv = jnp.where(hit, cvec, sv)
                sbase = jnp.where(hit, base, sbase)
                stot = jnp.where(hit, tot, stot)
                found = jnp.where(hit, 1, found)
                return (tot + csum, found, sv, sbase, stot)

            _, _, sv, sbase, stot = ph1

            rc = lax.rev(sv, (0,))
            cum = stot + plsc.cumsum(rc)
            ge = cum >= target
            ge2 = (cum - rc) >= target
            popc = jnp.max(plsc.all_reduce_population_count(ge))
            h = sbase + popc - 1
            cab = stot + jnp.sum(jnp.where(ge, 0, rc))
            cnt_at = jnp.sum(jnp.where(ge, rc, 0)) - \
                jnp.sum(jnp.where(ge2, rc, 0))
            return h, cab, cnt_at

        def do_row(j, carry):
            row = wid * _RPW + j
            pltpu.sync_copy(x_hbm.at[row], xv)

            @plsc.parallel_loop(0, _CHUNKS, unroll=8)
            def _p1(i):
                xb = xv[pl.ds(i * _L, _L)]
                sk = sortkey(lax.bitcast_convert_type(xb, jnp.int32))
                plsc.addupdate_scatter(hist, [(sk >> 20) + 2048], ones16)

            h1, cab1, _ = walk(_H // _L, jnp.int32(_K))
            pfx1 = h1 - 2048
            r1 = _K - cab1

            @plsc.parallel_loop(0, _CHUNKS, unroll=4, carry=jnp.int32(0))
            def pk(i, off):
                xb = xv[pl.ds(i * _L, _L)]
                sk = sortkey(lax.bitcast_convert_type(xb, jnp.int32))
                m = (sk >> 20) >= pfx1
                plsc.store_compressed(csk.at[pl.ds(off, _L)], sk, mask=m)
                plsc.store_compressed(cix.at[pl.ds(off, _L)],
                                      lanes + i * _L, mask=m)
                return off + jnp.max(plsc.all_reduce_population_count(m))

            cnt = pk
            nch = (cnt + _L - 1) // _L

            def refine(pshift, pfx, bshift, bmask, nbins, target):
                @plsc.parallel_loop(0, nch)
                def _rf(i):
                    base = i * _L
                    sk = csk[pl.ds(base, _L)]
                    m = jnp.logical_and(lanes < (cnt - base),
                                        (sk >> pshift) == pfx)
                    plsc.addupdate_scatter(hist, [(sk >> bshift) & bmask],
                                           ones16, mask=m)

                return walk(nbins // _L, target)

            b2a, cabA, _ = refine(20, pfx1, 12, 0xFF, 256, r1)
            pfx_a = (pfx1 << 8) | b2a
            r2a = r1 - cabA
            b2b, cabB, _ = refine(12, pfx_a, 4, 0xFF, 256, r2a)
            pfx_b = (pfx_a << 8) | b2b
            r2b = r2a - cabB
            b3, cabC, cnt_at = refine(4, pfx_b, 0, 0xF, 16, r2b)
            t = (pfx_b << 4) | b3
            tie_budget = r2b - cabC

            @plsc.parallel_loop(0, nch, carry=(jnp.int32(0), jnp.int32(0)))
            def sel(i, c):
                off, used = c
                base = i * _L
                sk = csk[pl.ds(base, _L)]
                ix = cix[pl.ds(base, _L)]
                valid = lanes < (cnt - base)
                gt = jnp.logical_and(valid, sk > t)
                eq = jnp.logical_and(valid, sk == t)
                eq_i = jnp.where(eq, 1, 0)
                tie_rank = used + plsc.cumsum(eq_i)
                keep = gt | jnp.logical_and(eq, tie_rank <= tie_budget)
                plsc.store_compressed(ksk.at[pl.ds(off, _L)], sk, mask=keep)
                plsc.store_compressed(kix.at[pl.ds(off, _L)], ix, mask=keep)
                off = off + jnp.max(plsc.all_reduce_population_count(keep))
                return (off, jnp.max(tie_rank))

            @plsc.parallel_loop(0, _K // _L, unroll=4)
            def _emit(i):
                sk = ksk[pl.ds(i * _L, _L)]
                ix = kix[pl.ds(i * _L, _L)]
                vals = lax.bitcast_convert_type(sortkey(sk), jnp.float32)
                plsc.store_scatter(outv, [ix], vals)

            pltpu.sync_copy(outv, out_hbm.at[row])

            @plsc.parallel_loop(0, _K // _L, unroll=4)
            def _unemit(i):
                ix = kix[pl.ds(i * _L, _L)]
                plsc.store_scatter(outv, [ix], zf16)

            return carry

        lax.fori_loop(0, _RPW, do_row, 0)

    return wta(x)
